# baseline (speedup 1.0000x reference)
import jax
import jax.numpy as jnp
from jax.experimental import pallas as pl

BM = 400


def _xw_kernel(x_ref, w_ref, o_ref):
    o_ref[...] = jnp.dot(x_ref[...], w_ref[...].T,
                         preferred_element_type=jnp.float32)


def _pass_b_kernel(adj_ref, t_ref, b1_ref, w2_ref, u_ref):
    h = jnp.dot(adj_ref[...], t_ref[...], preferred_element_type=jnp.float32)
    h = jnp.maximum(h + b1_ref[...], 0.0)
    u_ref[...] = jnp.dot(h, w2_ref[...].T, preferred_element_type=jnp.float32)


def _pass_c_kernel(adj_ref, u_ref, b2_ref, o_ref):
    z = jnp.dot(adj_ref[...], u_ref[...], preferred_element_type=jnp.float32)
    z = z + b2_ref[...]
    m = jnp.max(z, axis=1, keepdims=True)
    e = z - m
    lse = jnp.log(jnp.sum(jnp.exp(e), axis=1, keepdims=True))
    o_ref[...] = e - lse


@jax.jit
def kernel(x, adj, W1, b1, W2, b2):
    in_f = x.shape[1]
    hid = W1.shape[0]
    out_f = W2.shape[0]
    n = adj.shape[0]
    grid = (n // BM,)

    t = pl.pallas_call(
        _xw_kernel,
        out_shape=jax.ShapeDtypeStruct((n, hid), jnp.float32),
        in_specs=[
            pl.BlockSpec((n, in_f), lambda: (0, 0)),
            pl.BlockSpec((hid, in_f), lambda: (0, 0)),
        ],
        out_specs=pl.BlockSpec((n, hid), lambda: (0, 0)),
    )(x, W1)

    u = pl.pallas_call(
        _pass_b_kernel,
        grid=grid,
        out_shape=jax.ShapeDtypeStruct((n, out_f), jnp.float32),
        in_specs=[
            pl.BlockSpec((BM, n), lambda i: (i, 0)),
            pl.BlockSpec((n, hid), lambda i: (0, 0)),
            pl.BlockSpec((hid,), lambda i: (0,)),
            pl.BlockSpec((out_f, hid), lambda i: (0, 0)),
        ],
        out_specs=pl.BlockSpec((BM, out_f), lambda i: (i, 0)),
    )(adj, t, b1, W2)

    out = pl.pallas_call(
        _pass_c_kernel,
        grid=grid,
        out_shape=jax.ShapeDtypeStruct((n, out_f), jnp.float32),
        in_specs=[
            pl.BlockSpec((BM, n), lambda i: (i, 0)),
            pl.BlockSpec((n, out_f), lambda i: (0, 0)),
            pl.BlockSpec((out_f,), lambda i: (0,)),
        ],
        out_specs=pl.BlockSpec((BM, out_f), lambda i: (i, 0)),
    )(adj, u, b2)

    return out
